# baseline (speedup 1.0000x reference)
import functools

import jax
import jax.numpy as jnp
from jax import lax
from jax.experimental import pallas as pl
from jax.experimental.pallas import tpu as pltpu
from jax.experimental.pallas import tpu_sc as plsc

N = 10000
NPAD = 10112
GARBAGE_ROW = 10016
E = 320000
NW = 32
SUB = 256
KSUB = 40
KSUB2 = 80
EPAD = NW * KSUB * SUB
RPT = NPAD // 16
NG = 256
BN = 2528
GRID = NPAD // BN

_mesh = plsc.VectorSubcoreMesh(
    core_axis_name="c", subcore_axis_name="s", num_cores=2, num_subcores=16)
_sc_params = pltpu.CompilerParams(use_tc_tiling_on_sc=False)


NBUF = 4


def _ring_loop(g_ref, src_v, dst_v, rows, agg_sh, gsems, ssems, ksub):
  for b in range(NBUF):
    pltpu.async_copy(g_ref.at[src_v.at[b]], rows[b], gsems[b])

  def body(i, carry):
    for b in range(NBUF):
      j = NBUF * i + b
      pltpu.make_async_copy(g_ref.at[src_v.at[j]], rows[b], gsems[b]).wait()
      pltpu.async_copy(rows[b], agg_sh.at[dst_v.at[j]], ssems[b], add=True)
      nb = (b + 1) % NBUF

      @pl.when(jnp.logical_and(j >= NBUF - 1, j + 1 < ksub))
      def _():
        pltpu.make_async_copy(rows[nb], agg_sh.at[dst_v.at[0]],
                              ssems[nb]).wait()
        pltpu.async_copy(g_ref.at[src_v.at[j + 1]], rows[nb], gsems[nb])

    return carry

  lax.fori_loop(0, ksub // NBUF, body, 0)
  for b in range(NBUF):
    pltpu.make_async_copy(rows[b], agg_sh.at[dst_v.at[0]], ssems[b]).wait()


def _make_edge_agg(D):
  @functools.partial(
      pl.kernel,
      out_type=jax.ShapeDtypeStruct((2, NPAD, D), jnp.float32),
      mesh=_mesh,
      scratch_types=[
          pltpu.VMEM((KSUB, SUB), jnp.int32),
          pltpu.VMEM((KSUB, SUB), jnp.int32),
          [pltpu.VMEM((SUB, D), jnp.float32)] * NBUF,
          pltpu.VMEM_SHARED((NPAD, D), jnp.float32),
          [pltpu.SemaphoreType.DMA] * NBUF,
          [pltpu.SemaphoreType.DMA] * NBUF,
      ],
      compiler_params=_sc_params,
  )
  def k(g_hbm, srcw, dstw, zeros_hbm, out_hbm, src_v, dst_v, rows, agg_sh,
        gsem, ssem):
    c = lax.axis_index("c")
    s = lax.axis_index("s")
    w = c * 16 + s
    base = s * RPT
    pltpu.sync_copy(zeros_hbm.at[pl.ds(base, RPT)], agg_sh.at[pl.ds(base, RPT)])
    pltpu.sync_copy(srcw.at[w], src_v)
    pltpu.sync_copy(dstw.at[w], dst_v)
    plsc.subcore_barrier()
    _ring_loop(g_hbm, src_v, dst_v, rows, agg_sh, gsem, ssem, KSUB)
    plsc.subcore_barrier()
    pltpu.sync_copy(agg_sh.at[pl.ds(base, RPT)],
                    out_hbm.at[c, pl.ds(base, RPT)])

  return k


@functools.partial(
    pl.kernel,
    out_type=jax.ShapeDtypeStruct((2, NPAD, 64), jnp.float32),
    mesh=_mesh,
    scratch_types=[
        pltpu.VMEM((KSUB2 // 2, SUB), jnp.int32),
        pltpu.VMEM((KSUB2 // 2, SUB), jnp.int32),
        [pltpu.VMEM((SUB, 64), jnp.float32)] * NBUF,
        pltpu.VMEM_SHARED((NPAD, 64), jnp.float32),
        [pltpu.SemaphoreType.DMA] * NBUF,
        [pltpu.SemaphoreType.DMA] * NBUF,
    ],
    compiler_params=_sc_params,
)
def _agg_colsplit(g_hbm, srcw, dstw, zeros_hbm, out_hbm, src_v, dst_v, rows,
                  agg_sh, gsem, ssem):
  c = lax.axis_index("c")
  s = lax.axis_index("s")
  base = s * RPT
  half = KSUB2 // 2
  g_my = g_hbm.at[c]
  pltpu.sync_copy(zeros_hbm.at[pl.ds(base, RPT)], agg_sh.at[pl.ds(base, RPT)])
  plsc.subcore_barrier()
  for h in range(2):
    pltpu.sync_copy(srcw.at[s, pl.ds(h * half, half)], src_v)
    pltpu.sync_copy(dstw.at[s, pl.ds(h * half, half)], dst_v)
    _ring_loop(g_my, src_v, dst_v, rows, agg_sh, gsem, ssem, half)
  plsc.subcore_barrier()
  pltpu.sync_copy(agg_sh.at[pl.ds(base, RPT)], out_hbm.at[c, pl.ds(base, RPT)])


@functools.partial(
    pl.kernel,
    out_type=jax.ShapeDtypeStruct((2, NPAD, 16), jnp.float32),
    mesh=_mesh,
    scratch_types=[
        pltpu.VMEM((KSUB, SUB), jnp.int32),
        pltpu.VMEM((SUB, 16), jnp.float32),
        pltpu.VMEM_SHARED((NPAD, 16), jnp.float32),
        pltpu.SemaphoreType.DMA,
    ],
    compiler_params=_sc_params,
)
def _deg_kernel(dstw, zeros_hbm, ones_hbm, out_hbm, dst_v, ones_v, agg_sh,
                ssem):
  c = lax.axis_index("c")
  s = lax.axis_index("s")
  w = c * 16 + s
  base = s * RPT
  pltpu.sync_copy(zeros_hbm.at[pl.ds(base, RPT)], agg_sh.at[pl.ds(base, RPT)])
  pltpu.sync_copy(dstw.at[w], dst_v)
  pltpu.sync_copy(ones_hbm, ones_v)
  plsc.subcore_barrier()

  def body(i, carry):
    for b in range(8):
      pltpu.async_copy(ones_v, agg_sh.at[dst_v.at[8 * i + b]], ssem, add=True)
    for b in range(8):
      pltpu.make_async_copy(ones_v, agg_sh.at[dst_v.at[0]], ssem).wait()
    return carry

  lax.fori_loop(0, KSUB // 8, body, 0)
  plsc.subcore_barrier()
  pltpu.sync_copy(agg_sh.at[pl.ds(base, RPT)], out_hbm.at[c, pl.ds(base, RPT)])


_agg16 = _make_edge_agg(16)


def _row_spec(width):
  return pl.BlockSpec((BN, width), lambda i: (i, 0))


def _full_spec(shape):
  return pl.BlockSpec(shape, lambda i: tuple(0 for _ in shape))


def _t1_body(d0, d1, xp, dinv_out, xs_out):
  deg = d0[...][:, 0:1] + d1[...][:, 0:1] + 1.0
  dinv = 1.0 / jnp.sqrt(deg)
  dinv_out[...] = jnp.broadcast_to(dinv, (BN, 16))
  xs_out[...] = xp[...] * dinv


def _t3_body(a0, a1, xs, dinv, w1, b1, w2, g2_out):
  y = dinv[...] * (a0[...] + a1[...] + xs[...])
  h1 = jnp.maximum(
      jnp.dot(y, w1[...], preferred_element_type=jnp.float32,
      precision=lax.Precision.HIGHEST) + b1[...], 0.0)
  g2 = jnp.dot(h1, w2[...], preferred_element_type=jnp.float32,
      precision=lax.Precision.HIGHEST)
  g2 = dinv[...][:, 0:1] * g2
  g2_out[0, :, :] = g2[:, :64]
  g2_out[1, :, :] = g2[:, 64:]


def _t5_body(ga, g2, dinv, b2, w3, wl, z16_out):
  dv = dinv[...][:, 0:1]
  agg = jnp.concatenate([ga[0], ga[1]], axis=1)
  g2full = jnp.concatenate([g2[0], g2[1]], axis=1)
  h2 = jnp.maximum(dv * (agg + g2full) + b2[...], 0.0)
  w3l = jnp.dot(w3[...], wl[...], preferred_element_type=jnp.float32,
      precision=lax.Precision.HIGHEST)
  z = dv * jnp.dot(h2, w3l, preferred_element_type=jnp.float32,
      precision=lax.Precision.HIGHEST)
  z16_out[...] = jnp.broadcast_to(z, (BN, 16))


def _t7_body(za, zb, z16, dinv, bt, b3, wl, bl, out, pooled_acc, counts_acc):
  i = pl.program_id(0)
  s = dinv[...][:, 0:1] * (
      za[...][:, 0:1] + zb[...][:, 0:1] + z16[...][:, 0:1])
  oh = (bt[...] == lax.broadcasted_iota(jnp.int32, (BN, NG), 1)
        ).astype(jnp.float32)
  dims = (((0,), (0,)), ((), ()))
  pb = lax.dot_general(oh, s, dims, preferred_element_type=jnp.float32,
      precision=lax.Precision.HIGHEST)
  cb = lax.dot_general(oh, jnp.ones((BN, 1), jnp.float32), dims,
                       preferred_element_type=jnp.float32,
      precision=lax.Precision.HIGHEST)

  @pl.when(i == 0)
  def _():
    pooled_acc[...] = pb
    counts_acc[...] = cb

  @pl.when(i > 0)
  def _():
    pooled_acc[...] = pooled_acc[...] + pb
    counts_acc[...] = counts_acc[...] + cb

  @pl.when(i == GRID - 1)
  def _():
    const = jnp.dot(b3[...], wl[...],
                    preferred_element_type=jnp.float32,
      precision=lax.Precision.HIGHEST) + bl[...]
    out[...] = pooled_acc[...] / jnp.clip(counts_acc[...], 1.0, None) + const


def kernel(x, edge_index, batch, W1, b1, W2, b2, W3, b3, Wl, bl):
  f32 = jnp.float32
  src = edge_index[0]
  dst = edge_index[1]
  npad_e = EPAD - E
  srcp = jnp.concatenate([src, jnp.zeros((npad_e,), jnp.int32)])
  dstp = jnp.concatenate([dst, jnp.full((npad_e,), GARBAGE_ROW, jnp.int32)])
  srcw = srcp.reshape(NW, KSUB, SUB)
  dstw = dstp.reshape(NW, KSUB, SUB)
  srcw2 = srcp.reshape(16, KSUB2, SUB)
  dstw2 = dstp.reshape(16, KSUB2, SUB)

  xpad = jnp.zeros((NPAD, 16), f32).at[:N, :9].set(x)
  zeros16 = jnp.zeros((NPAD, 16), f32)
  zeros64 = jnp.zeros((NPAD, 64), f32)
  ones16 = jnp.ones((SUB, 16), f32)
  batchpad = jnp.full((NPAD, 1), NG, jnp.int32).at[:N, 0].set(batch)
  w1p = jnp.zeros((16, 128), f32).at[:9, :].set(W1)

  degp = _deg_kernel(dstw, zeros16, ones16)

  dinv16, xs16 = pl.pallas_call(
      _t1_body,
      grid=(GRID,),
      in_specs=[_row_spec(16), _row_spec(16), _row_spec(16)],
      out_specs=[_row_spec(16), _row_spec(16)],
      out_shape=[
          jax.ShapeDtypeStruct((NPAD, 16), f32),
          jax.ShapeDtypeStruct((NPAD, 16), f32),
      ],
  )(degp[0], degp[1], xpad)

  xa = _agg16(xs16, srcw, dstw, zeros16)

  g2 = pl.pallas_call(
      _t3_body,
      grid=(GRID,),
      in_specs=[
          _row_spec(16), _row_spec(16), _row_spec(16), _row_spec(16),
          _full_spec((16, 128)), _full_spec((1, 128)), _full_spec((128, 128)),
      ],
      out_specs=pl.BlockSpec((2, BN, 64), lambda i: (0, i, 0)),
      out_shape=jax.ShapeDtypeStruct((2, NPAD, 64), f32),
  )(xa[0], xa[1], xs16, dinv16, w1p, b1.reshape(1, 128), W2)

  ga = _agg_colsplit(g2, srcw2, dstw2, zeros64)

  z16 = pl.pallas_call(
      _t5_body,
      grid=(GRID,),
      in_specs=[
          pl.BlockSpec((2, BN, 64), lambda i: (0, i, 0)),
          pl.BlockSpec((2, BN, 64), lambda i: (0, i, 0)),
          _row_spec(16),
          _full_spec((1, 128)), _full_spec((128, 128)), _full_spec((128, 1)),
      ],
      out_specs=_row_spec(16),
      out_shape=jax.ShapeDtypeStruct((NPAD, 16), f32),
  )(ga, g2, dinv16, b2.reshape(1, 128), W3, Wl)

  za = _agg16(z16, srcw, dstw, zeros16)

  out = pl.pallas_call(
      _t7_body,
      grid=(GRID,),
      in_specs=[
          _row_spec(16), _row_spec(16), _row_spec(16), _row_spec(16),
          pl.BlockSpec((BN, 1), lambda i: (i, 0)),
          _full_spec((1, 128)), _full_spec((128, 1)), _full_spec((1, 1)),
      ],
      out_specs=pl.BlockSpec((NG, 1), lambda i: (0, 0)),
      out_shape=jax.ShapeDtypeStruct((NG, 1), f32),
      scratch_shapes=[
          pltpu.VMEM((NG, 1), f32),
          pltpu.VMEM((NG, 1), f32),
      ],
  )(za[0], za[1], z16, dinv16, batchpad, b3.reshape(1, 128), Wl,
    bl.reshape(1, 1))
  return out


# speedup vs baseline: 1.1044x; 1.1044x over previous
import functools

import jax
import jax.numpy as jnp
from jax import lax
from jax.experimental import pallas as pl
from jax.experimental.pallas import tpu as pltpu
from jax.experimental.pallas import tpu_sc as plsc

N = 10000
NPAD = 10112
GARBAGE_ROW = 10016
E = 320000
NW = 32
SUB = 256
KSUB = 40
KSUB2 = 80
EPAD = NW * KSUB * SUB
RPT = NPAD // 16
NG = 256
BN = 2528
GRID = NPAD // BN

_mesh = plsc.VectorSubcoreMesh(
    core_axis_name="c", subcore_axis_name="s", num_cores=2, num_subcores=16)
_sc_params = pltpu.CompilerParams(use_tc_tiling_on_sc=False)


NBUF = 2


def _ring_loop(g_ref, src_v, dst_v, rows, agg_sh, gsems, ssems, ksub):
  rows_a, rows_b = rows
  sem_a, sem_b = gsems
  del ssems
  pltpu.async_copy(g_ref.at[src_v.at[0]], rows_a, sem_a)
  pltpu.async_copy(g_ref.at[src_v.at[1]], rows_b, sem_b)

  def body(i, carry):
    j0 = 2 * i
    pltpu.make_async_copy(g_ref.at[src_v.at[j0]], rows_a, sem_a).wait()
    pltpu.sync_copy(rows_a, agg_sh.at[dst_v.at[j0]], add=True)

    @pl.when(j0 + 2 < ksub)
    def _():
      pltpu.async_copy(g_ref.at[src_v.at[j0 + 2]], rows_a, sem_a)

    pltpu.make_async_copy(g_ref.at[src_v.at[j0 + 1]], rows_b, sem_b).wait()
    pltpu.sync_copy(rows_b, agg_sh.at[dst_v.at[j0 + 1]], add=True)

    @pl.when(j0 + 3 < ksub)
    def _():
      pltpu.async_copy(g_ref.at[src_v.at[j0 + 3]], rows_b, sem_b)

    return carry

  lax.fori_loop(0, ksub // 2, body, 0)


def _make_edge_agg(D):
  @functools.partial(
      pl.kernel,
      out_type=jax.ShapeDtypeStruct((2, NPAD, D), jnp.float32),
      mesh=_mesh,
      scratch_types=[
          pltpu.VMEM((KSUB, SUB), jnp.int32),
          pltpu.VMEM((KSUB, SUB), jnp.int32),
          [pltpu.VMEM((SUB, D), jnp.float32)] * NBUF,
          pltpu.VMEM_SHARED((NPAD, D), jnp.float32),
          [pltpu.SemaphoreType.DMA] * NBUF,
          [pltpu.SemaphoreType.DMA] * NBUF,
      ],
      compiler_params=_sc_params,
  )
  def k(g_hbm, srcw, dstw, zeros_hbm, out_hbm, src_v, dst_v, rows, agg_sh,
        gsem, ssem):
    c = lax.axis_index("c")
    s = lax.axis_index("s")
    w = c * 16 + s
    base = s * RPT
    pltpu.sync_copy(zeros_hbm.at[pl.ds(base, RPT)], agg_sh.at[pl.ds(base, RPT)])
    pltpu.sync_copy(srcw.at[w], src_v)
    pltpu.sync_copy(dstw.at[w], dst_v)
    plsc.subcore_barrier()
    _ring_loop(g_hbm, src_v, dst_v, rows, agg_sh, gsem, ssem, KSUB)
    plsc.subcore_barrier()
    pltpu.sync_copy(agg_sh.at[pl.ds(base, RPT)],
                    out_hbm.at[c, pl.ds(base, RPT)])

  return k


@functools.partial(
    pl.kernel,
    out_type=jax.ShapeDtypeStruct((2, NPAD, 64), jnp.float32),
    mesh=_mesh,
    scratch_types=[
        pltpu.VMEM((KSUB2, SUB), jnp.int32),
        pltpu.VMEM((KSUB2, SUB), jnp.int32),
        [pltpu.VMEM((SUB, 64), jnp.float32)] * NBUF,
        pltpu.VMEM_SHARED((NPAD, 64), jnp.float32),
        [pltpu.SemaphoreType.DMA] * NBUF,
        [pltpu.SemaphoreType.DMA] * NBUF,
    ],
    compiler_params=_sc_params,
)
def _agg_colsplit(g_hbm, srcw, dstw, zeros_hbm, out_hbm, src_v, dst_v, rows,
                  agg_sh, gsem, ssem):
  c = lax.axis_index("c")
  s = lax.axis_index("s")
  base = s * RPT
  g_my = g_hbm.at[c]
  pltpu.sync_copy(zeros_hbm.at[pl.ds(base, RPT)], agg_sh.at[pl.ds(base, RPT)])
  pltpu.sync_copy(srcw.at[s], src_v)
  pltpu.sync_copy(dstw.at[s], dst_v)
  plsc.subcore_barrier()
  _ring_loop(g_my, src_v, dst_v, rows, agg_sh, gsem, ssem, KSUB2)
  plsc.subcore_barrier()
  pltpu.sync_copy(agg_sh.at[pl.ds(base, RPT)], out_hbm.at[c, pl.ds(base, RPT)])


@functools.partial(
    pl.kernel,
    out_type=jax.ShapeDtypeStruct((2, NPAD, 16), jnp.float32),
    mesh=_mesh,
    scratch_types=[
        pltpu.VMEM((KSUB, SUB), jnp.int32),
        pltpu.VMEM((SUB, 16), jnp.float32),
        pltpu.VMEM_SHARED((NPAD, 16), jnp.float32),
        pltpu.SemaphoreType.DMA,
    ],
    compiler_params=_sc_params,
)
def _deg_kernel(dstw, zeros_hbm, ones_hbm, out_hbm, dst_v, ones_v, agg_sh,
                ssem):
  c = lax.axis_index("c")
  s = lax.axis_index("s")
  w = c * 16 + s
  base = s * RPT
  pltpu.sync_copy(zeros_hbm.at[pl.ds(base, RPT)], agg_sh.at[pl.ds(base, RPT)])
  pltpu.sync_copy(dstw.at[w], dst_v)
  pltpu.sync_copy(ones_hbm, ones_v)
  plsc.subcore_barrier()

  def body(i, carry):
    for b in range(8):
      pltpu.async_copy(ones_v, agg_sh.at[dst_v.at[8 * i + b]], ssem, add=True)
    for b in range(8):
      pltpu.make_async_copy(ones_v, agg_sh.at[dst_v.at[0]], ssem).wait()
    return carry

  lax.fori_loop(0, KSUB // 8, body, 0)
  plsc.subcore_barrier()
  pltpu.sync_copy(agg_sh.at[pl.ds(base, RPT)], out_hbm.at[c, pl.ds(base, RPT)])


_agg16 = _make_edge_agg(16)


def _row_spec(width):
  return pl.BlockSpec((BN, width), lambda i: (i, 0))


def _full_spec(shape):
  return pl.BlockSpec(shape, lambda i: tuple(0 for _ in shape))


def _t1_body(d0, d1, xp, dinv_out, xs_out):
  deg = d0[...][:, 0:1] + d1[...][:, 0:1] + 1.0
  dinv = 1.0 / jnp.sqrt(deg)
  dinv_out[...] = jnp.broadcast_to(dinv, (BN, 16))
  xs_out[...] = xp[...] * dinv


def _t3_body(a0, a1, xs, dinv, w1, b1, w2, g2_out):
  y = dinv[...] * (a0[...] + a1[...] + xs[...])
  h1 = jnp.maximum(
      jnp.dot(y, w1[...], preferred_element_type=jnp.float32) + b1[...], 0.0)
  g2 = jnp.dot(h1, w2[...], preferred_element_type=jnp.float32)
  g2 = dinv[...][:, 0:1] * g2
  g2_out[0, :, :] = g2[:, :64]
  g2_out[1, :, :] = g2[:, 64:]


def _t5_body(ga, g2, dinv, b2, w3, wl, z16_out):
  dv = dinv[...][:, 0:1]
  agg = jnp.concatenate([ga[0], ga[1]], axis=1)
  g2full = jnp.concatenate([g2[0], g2[1]], axis=1)
  h2 = jnp.maximum(dv * (agg + g2full) + b2[...], 0.0)
  w3l = jnp.dot(w3[...], wl[...], preferred_element_type=jnp.float32)
  z = dv * jnp.dot(h2, w3l, preferred_element_type=jnp.float32)
  z16_out[...] = jnp.broadcast_to(z, (BN, 16))


def _t7_body(za, zb, z16, dinv, bt, b3, wl, bl, out, pooled_acc, counts_acc):
  i = pl.program_id(0)
  s = dinv[...][:, 0:1] * (
      za[...][:, 0:1] + zb[...][:, 0:1] + z16[...][:, 0:1])
  oh = (bt[...] == lax.broadcasted_iota(jnp.int32, (BN, NG), 1)
        ).astype(jnp.float32)
  dims = (((0,), (0,)), ((), ()))
  pb = lax.dot_general(oh, s, dims, preferred_element_type=jnp.float32)
  cb = lax.dot_general(oh, jnp.ones((BN, 1), jnp.float32), dims,
                       preferred_element_type=jnp.float32)

  @pl.when(i == 0)
  def _():
    pooled_acc[...] = pb
    counts_acc[...] = cb

  @pl.when(i > 0)
  def _():
    pooled_acc[...] = pooled_acc[...] + pb
    counts_acc[...] = counts_acc[...] + cb

  @pl.when(i == GRID - 1)
  def _():
    const = jnp.dot(b3[...], wl[...],
                    preferred_element_type=jnp.float32) + bl[...]
    out[...] = pooled_acc[...] / jnp.clip(counts_acc[...], 1.0, None) + const


def kernel(x, edge_index, batch, W1, b1, W2, b2, W3, b3, Wl, bl):
  f32 = jnp.float32
  src = edge_index[0]
  dst = edge_index[1]
  npad_e = EPAD - E
  srcp = jnp.concatenate([src, jnp.zeros((npad_e,), jnp.int32)])
  dstp = jnp.concatenate([dst, jnp.full((npad_e,), GARBAGE_ROW, jnp.int32)])
  srcw = srcp.reshape(NW, KSUB, SUB)
  dstw = dstp.reshape(NW, KSUB, SUB)
  srcw2 = srcp.reshape(16, KSUB2, SUB)
  dstw2 = dstp.reshape(16, KSUB2, SUB)

  xpad = jnp.zeros((NPAD, 16), f32).at[:N, :9].set(x)
  zeros16 = jnp.zeros((NPAD, 16), f32)
  zeros64 = jnp.zeros((NPAD, 64), f32)
  ones16 = jnp.ones((SUB, 16), f32)
  batchpad = jnp.full((NPAD, 1), NG, jnp.int32).at[:N, 0].set(batch)
  w1p = jnp.zeros((16, 128), f32).at[:9, :].set(W1)

  degp = _deg_kernel(dstw, zeros16, ones16)

  dinv16, xs16 = pl.pallas_call(
      _t1_body,
      grid=(GRID,),
      in_specs=[_row_spec(16), _row_spec(16), _row_spec(16)],
      out_specs=[_row_spec(16), _row_spec(16)],
      out_shape=[
          jax.ShapeDtypeStruct((NPAD, 16), f32),
          jax.ShapeDtypeStruct((NPAD, 16), f32),
      ],
  )(degp[0], degp[1], xpad)

  xa = _agg16(xs16, srcw, dstw, zeros16)

  g2 = pl.pallas_call(
      _t3_body,
      grid=(GRID,),
      in_specs=[
          _row_spec(16), _row_spec(16), _row_spec(16), _row_spec(16),
          _full_spec((16, 128)), _full_spec((1, 128)), _full_spec((128, 128)),
      ],
      out_specs=pl.BlockSpec((2, BN, 64), lambda i: (0, i, 0)),
      out_shape=jax.ShapeDtypeStruct((2, NPAD, 64), f32),
  )(xa[0], xa[1], xs16, dinv16, w1p, b1.reshape(1, 128), W2)

  ga = _agg_colsplit(g2, srcw2, dstw2, zeros64)

  z16 = pl.pallas_call(
      _t5_body,
      grid=(GRID,),
      in_specs=[
          pl.BlockSpec((2, BN, 64), lambda i: (0, i, 0)),
          pl.BlockSpec((2, BN, 64), lambda i: (0, i, 0)),
          _row_spec(16),
          _full_spec((1, 128)), _full_spec((128, 128)), _full_spec((128, 1)),
      ],
      out_specs=_row_spec(16),
      out_shape=jax.ShapeDtypeStruct((NPAD, 16), f32),
  )(ga, g2, dinv16, b2.reshape(1, 128), W3, Wl)

  za = _agg16(z16, srcw, dstw, zeros16)

  out = pl.pallas_call(
      _t7_body,
      grid=(GRID,),
      in_specs=[
          _row_spec(16), _row_spec(16), _row_spec(16), _row_spec(16),
          pl.BlockSpec((BN, 1), lambda i: (i, 0)),
          _full_spec((1, 128)), _full_spec((128, 1)), _full_spec((1, 1)),
      ],
      out_specs=pl.BlockSpec((NG, 1), lambda i: (0, 0)),
      out_shape=jax.ShapeDtypeStruct((NG, 1), f32),
      scratch_shapes=[
          pltpu.VMEM((NG, 1), f32),
          pltpu.VMEM((NG, 1), f32),
      ],
  )(za[0], za[1], z16, dinv16, batchpad, b3.reshape(1, 128), Wl,
    bl.reshape(1, 1))
  return out


# speedup vs baseline: 1.1306x; 1.0237x over previous
import functools

import jax
import jax.numpy as jnp
from jax import lax
from jax.experimental import pallas as pl
from jax.experimental.pallas import tpu as pltpu
from jax.experimental.pallas import tpu_sc as plsc

N = 10000
NPAD = 10112
GARBAGE_ROW = 10016
E = 320000
NW = 32
SUB = 512
KSUB = 20
SUBC = 256
KSUB2 = 80
EPAD = NW * KSUB * SUB
RPT = NPAD // 16
NG = 256
BN = 2528
GRID = NPAD // BN

_mesh = plsc.VectorSubcoreMesh(
    core_axis_name="c", subcore_axis_name="s", num_cores=2, num_subcores=16)
_sc_params = pltpu.CompilerParams(use_tc_tiling_on_sc=False)


NBUF = 2


def _ring_loop(g_ref, src_v, dst_v, rows, agg_sh, gsems, ssems, ksub):
  rows_a, rows_b = rows
  sem_a, sem_b = gsems
  del ssems
  pltpu.async_copy(g_ref.at[src_v.at[0]], rows_a, sem_a)
  pltpu.async_copy(g_ref.at[src_v.at[1]], rows_b, sem_b)

  def body(i, carry):
    j0 = 2 * i
    pltpu.make_async_copy(g_ref.at[src_v.at[j0]], rows_a, sem_a).wait()
    pltpu.sync_copy(rows_a, agg_sh.at[dst_v.at[j0]], add=True)

    @pl.when(j0 + 2 < ksub)
    def _():
      pltpu.async_copy(g_ref.at[src_v.at[j0 + 2]], rows_a, sem_a)

    pltpu.make_async_copy(g_ref.at[src_v.at[j0 + 1]], rows_b, sem_b).wait()
    pltpu.sync_copy(rows_b, agg_sh.at[dst_v.at[j0 + 1]], add=True)

    @pl.when(j0 + 3 < ksub)
    def _():
      pltpu.async_copy(g_ref.at[src_v.at[j0 + 3]], rows_b, sem_b)

    return carry

  lax.fori_loop(0, ksub // 2, body, 0)


def _make_edge_agg(D):
  @functools.partial(
      pl.kernel,
      out_type=jax.ShapeDtypeStruct((2, NPAD, D), jnp.float32),
      mesh=_mesh,
      scratch_types=[
          pltpu.VMEM((KSUB, SUB), jnp.int32),
          pltpu.VMEM((KSUB, SUB), jnp.int32),
          [pltpu.VMEM((SUB, D), jnp.float32)] * NBUF,
          pltpu.VMEM_SHARED((NPAD, D), jnp.float32),
          [pltpu.SemaphoreType.DMA] * NBUF,
          [pltpu.SemaphoreType.DMA] * NBUF,
      ],
      compiler_params=_sc_params,
  )
  def k(g_hbm, srcw, dstw, zeros_hbm, out_hbm, src_v, dst_v, rows, agg_sh,
        gsem, ssem):
    c = lax.axis_index("c")
    s = lax.axis_index("s")
    w = c * 16 + s
    base = s * RPT
    pltpu.sync_copy(zeros_hbm.at[pl.ds(base, RPT)], agg_sh.at[pl.ds(base, RPT)])
    pltpu.sync_copy(srcw.at[w], src_v)
    pltpu.sync_copy(dstw.at[w], dst_v)
    plsc.subcore_barrier()
    _ring_loop(g_hbm, src_v, dst_v, rows, agg_sh, gsem, ssem, KSUB)
    plsc.subcore_barrier()
    pltpu.sync_copy(agg_sh.at[pl.ds(base, RPT)],
                    out_hbm.at[c, pl.ds(base, RPT)])

  return k


@functools.partial(
    pl.kernel,
    out_type=jax.ShapeDtypeStruct((2, NPAD, 64), jnp.float32),
    mesh=_mesh,
    scratch_types=[
        pltpu.VMEM((KSUB2, SUBC), jnp.int32),
        pltpu.VMEM((KSUB2, SUBC), jnp.int32),
        [pltpu.VMEM((SUBC, 64), jnp.float32)] * NBUF,
        pltpu.VMEM_SHARED((NPAD, 64), jnp.float32),
        [pltpu.SemaphoreType.DMA] * NBUF,
        [pltpu.SemaphoreType.DMA] * NBUF,
    ],
    compiler_params=_sc_params,
)
def _agg_colsplit(g_hbm, srcw, dstw, zeros_hbm, out_hbm, src_v, dst_v, rows,
                  agg_sh, gsem, ssem):
  c = lax.axis_index("c")
  s = lax.axis_index("s")
  base = s * RPT
  g_my = g_hbm.at[c]
  pltpu.sync_copy(zeros_hbm.at[pl.ds(base, RPT)], agg_sh.at[pl.ds(base, RPT)])
  pltpu.sync_copy(srcw.at[s], src_v)
  pltpu.sync_copy(dstw.at[s], dst_v)
  plsc.subcore_barrier()
  _ring_loop(g_my, src_v, dst_v, rows, agg_sh, gsem, ssem, KSUB2)
  plsc.subcore_barrier()
  pltpu.sync_copy(agg_sh.at[pl.ds(base, RPT)], out_hbm.at[c, pl.ds(base, RPT)])


@functools.partial(
    pl.kernel,
    out_type=jax.ShapeDtypeStruct((2, NPAD, 16), jnp.float32),
    mesh=_mesh,
    scratch_types=[
        pltpu.VMEM((KSUB, SUB), jnp.int32),
        pltpu.VMEM((SUB, 16), jnp.float32),
        pltpu.VMEM_SHARED((NPAD, 16), jnp.float32),
        pltpu.SemaphoreType.DMA,
    ],
    compiler_params=_sc_params,
)
def _deg_kernel(dstw, zeros_hbm, ones_hbm, out_hbm, dst_v, ones_v, agg_sh,
                ssem):
  c = lax.axis_index("c")
  s = lax.axis_index("s")
  w = c * 16 + s
  base = s * RPT
  pltpu.sync_copy(zeros_hbm.at[pl.ds(base, RPT)], agg_sh.at[pl.ds(base, RPT)])
  pltpu.sync_copy(dstw.at[w], dst_v)
  pltpu.sync_copy(ones_hbm, ones_v)
  plsc.subcore_barrier()

  def body(i, carry):
    for b in range(4):
      pltpu.async_copy(ones_v, agg_sh.at[dst_v.at[4 * i + b]], ssem, add=True)
    for b in range(4):
      pltpu.make_async_copy(ones_v, agg_sh.at[dst_v.at[0]], ssem).wait()
    return carry

  lax.fori_loop(0, KSUB // 4, body, 0)
  plsc.subcore_barrier()
  pltpu.sync_copy(agg_sh.at[pl.ds(base, RPT)], out_hbm.at[c, pl.ds(base, RPT)])


_agg16 = _make_edge_agg(16)


def _row_spec(width):
  return pl.BlockSpec((BN, width), lambda i: (i, 0))


def _full_spec(shape):
  return pl.BlockSpec(shape, lambda i: tuple(0 for _ in shape))


def _t1_body(d0, d1, xp, dinv_out, xs_out):
  deg = d0[...][:, 0:1] + d1[...][:, 0:1] + 1.0
  dinv = 1.0 / jnp.sqrt(deg)
  dinv_out[...] = jnp.broadcast_to(dinv, (BN, 16))
  xs_out[...] = xp[...] * dinv


def _t3_body(a0, a1, xs, dinv, w1, b1, w2, g2_out):
  y = dinv[...] * (a0[...] + a1[...] + xs[...])
  h1 = jnp.maximum(
      jnp.dot(y, w1[...], preferred_element_type=jnp.float32) + b1[...], 0.0)
  g2 = jnp.dot(h1, w2[...], preferred_element_type=jnp.float32)
  g2 = dinv[...][:, 0:1] * g2
  g2_out[0, :, :] = g2[:, :64]
  g2_out[1, :, :] = g2[:, 64:]


def _t5_body(ga, g2, dinv, b2, w3, wl, z16_out):
  dv = dinv[...][:, 0:1]
  agg = jnp.concatenate([ga[0], ga[1]], axis=1)
  g2full = jnp.concatenate([g2[0], g2[1]], axis=1)
  h2 = jnp.maximum(dv * (agg + g2full) + b2[...], 0.0)
  w3l = jnp.dot(w3[...], wl[...], preferred_element_type=jnp.float32)
  z = dv * jnp.dot(h2, w3l, preferred_element_type=jnp.float32)
  z16_out[...] = jnp.broadcast_to(z, (BN, 16))


def _t7_body(za, zb, z16, dinv, bt, b3, wl, bl, out, pooled_acc, counts_acc):
  i = pl.program_id(0)
  s = dinv[...][:, 0:1] * (
      za[...][:, 0:1] + zb[...][:, 0:1] + z16[...][:, 0:1])
  oh = (bt[...] == lax.broadcasted_iota(jnp.int32, (BN, NG), 1)
        ).astype(jnp.float32)
  dims = (((0,), (0,)), ((), ()))
  pb = lax.dot_general(oh, s, dims, preferred_element_type=jnp.float32)
  cb = lax.dot_general(oh, jnp.ones((BN, 1), jnp.float32), dims,
                       preferred_element_type=jnp.float32)

  @pl.when(i == 0)
  def _():
    pooled_acc[...] = pb
    counts_acc[...] = cb

  @pl.when(i > 0)
  def _():
    pooled_acc[...] = pooled_acc[...] + pb
    counts_acc[...] = counts_acc[...] + cb

  @pl.when(i == GRID - 1)
  def _():
    const = jnp.dot(b3[...], wl[...],
                    preferred_element_type=jnp.float32) + bl[...]
    out[...] = pooled_acc[...] / jnp.clip(counts_acc[...], 1.0, None) + const


def kernel(x, edge_index, batch, W1, b1, W2, b2, W3, b3, Wl, bl):
  f32 = jnp.float32
  src = edge_index[0]
  dst = edge_index[1]
  npad_e = EPAD - E
  srcp = jnp.concatenate([src, jnp.zeros((npad_e,), jnp.int32)])
  dstp = jnp.concatenate([dst, jnp.full((npad_e,), GARBAGE_ROW, jnp.int32)])
  srcw = srcp.reshape(NW, KSUB, SUB)
  dstw = dstp.reshape(NW, KSUB, SUB)
  srcw2 = srcp.reshape(16, KSUB2, SUBC)
  dstw2 = dstp.reshape(16, KSUB2, SUBC)

  xpad = jnp.zeros((NPAD, 16), f32).at[:N, :9].set(x)
  zeros16 = jnp.zeros((NPAD, 16), f32)
  zeros64 = jnp.zeros((NPAD, 64), f32)
  ones16 = jnp.ones((SUB, 16), f32)
  batchpad = jnp.full((NPAD, 1), NG, jnp.int32).at[:N, 0].set(batch)
  w1p = jnp.zeros((16, 128), f32).at[:9, :].set(W1)

  degp = _deg_kernel(dstw, zeros16, ones16)

  dinv16, xs16 = pl.pallas_call(
      _t1_body,
      grid=(GRID,),
      in_specs=[_row_spec(16), _row_spec(16), _row_spec(16)],
      out_specs=[_row_spec(16), _row_spec(16)],
      out_shape=[
          jax.ShapeDtypeStruct((NPAD, 16), f32),
          jax.ShapeDtypeStruct((NPAD, 16), f32),
      ],
  )(degp[0], degp[1], xpad)

  xa = _agg16(xs16, srcw, dstw, zeros16)

  g2 = pl.pallas_call(
      _t3_body,
      grid=(GRID,),
      in_specs=[
          _row_spec(16), _row_spec(16), _row_spec(16), _row_spec(16),
          _full_spec((16, 128)), _full_spec((1, 128)), _full_spec((128, 128)),
      ],
      out_specs=pl.BlockSpec((2, BN, 64), lambda i: (0, i, 0)),
      out_shape=jax.ShapeDtypeStruct((2, NPAD, 64), f32),
  )(xa[0], xa[1], xs16, dinv16, w1p, b1.reshape(1, 128), W2)

  ga = _agg_colsplit(g2, srcw2, dstw2, zeros64)

  z16 = pl.pallas_call(
      _t5_body,
      grid=(GRID,),
      in_specs=[
          pl.BlockSpec((2, BN, 64), lambda i: (0, i, 0)),
          pl.BlockSpec((2, BN, 64), lambda i: (0, i, 0)),
          _row_spec(16),
          _full_spec((1, 128)), _full_spec((128, 128)), _full_spec((128, 1)),
      ],
      out_specs=_row_spec(16),
      out_shape=jax.ShapeDtypeStruct((NPAD, 16), f32),
  )(ga, g2, dinv16, b2.reshape(1, 128), W3, Wl)

  za = _agg16(z16, srcw, dstw, zeros16)

  out = pl.pallas_call(
      _t7_body,
      grid=(GRID,),
      in_specs=[
          _row_spec(16), _row_spec(16), _row_spec(16), _row_spec(16),
          pl.BlockSpec((BN, 1), lambda i: (i, 0)),
          _full_spec((1, 128)), _full_spec((128, 1)), _full_spec((1, 1)),
      ],
      out_specs=pl.BlockSpec((NG, 1), lambda i: (0, 0)),
      out_shape=jax.ShapeDtypeStruct((NG, 1), f32),
      scratch_shapes=[
          pltpu.VMEM((NG, 1), f32),
          pltpu.VMEM((NG, 1), f32),
      ],
  )(za[0], za[1], z16, dinv16, batchpad, b3.reshape(1, 128), Wl,
    bl.reshape(1, 1))
  return out
